# baseline (speedup 1.0000x reference)
import functools

import jax
import jax.numpy as jnp
from jax import lax
from jax.experimental import pallas as pl
from jax.experimental.pallas import tpu as pltpu
from jax.experimental.pallas import tpu_sc as plsc

B, L = 16384, 50
VOCAB, D = 1000000, 64
LP = 64
NC, NS = 2, 16
NW = NC * NS
BW = B // NW
CH = 128
NCHUNK = L * (BW // CH)
MM_BLK = 8192


def _mm_body(tt_ref, w_ref, b_ref, o_ref):
    tw = lax.dot_general(
        tt_ref[...], w_ref[...],
        dimension_numbers=(((0,), (0,)), ((), ())),
        preferred_element_type=jnp.float32,
    ) + b_ref[...]
    o_ref[:, 0:D] = tw
    o_ref[:, D:2 * D] = tw


def _table_times_w(table_t, W, b):
    return pl.pallas_call(
        _mm_body,
        grid=(pl.cdiv(VOCAB, MM_BLK),),
        in_specs=[
            pl.BlockSpec((D, MM_BLK), lambda i: (0, i)),
            pl.BlockSpec((D, D), lambda i: (0, 0)),
            pl.BlockSpec((1, D), lambda i: (0, 0)),
        ],
        out_specs=pl.BlockSpec((MM_BLK, 2 * D), lambda i: (i, 0)),
        out_shape=jax.ShapeDtypeStruct((VOCAB, 2 * D), jnp.float32),
    )(table_t, W, b.reshape(1, D))


def _gather_body(xp_hbm, tw_hbm, out_hbm, mask_hbm,
                 idx_v, bufA0, bufA1, bufT0, bufT1, mask_v, sem0, sem1):
    wid = lax.axis_index("s") * NC + lax.axis_index("c")
    b0 = wid * BW

    pltpu.sync_copy(xp_hbm.at[:, pl.ds(b0, BW)], idx_v)

    def start(t, buf, sem):
        l, c = t // 4, t % 4
        pltpu.make_async_copy(
            tw_hbm.at[idx_v.at[l, pl.ds(c * CH, CH)]], buf, sem).start()

    def finish(t, bufA, bufT, sem):
        l, c = t // 4, t % 4
        pltpu.make_async_copy(
            tw_hbm.at[idx_v.at[l, pl.ds(c * CH, CH)]], bufA, sem).wait()

        def drow(d, carry):
            cid = jnp.full((16,), d, jnp.int32)
            for j in range(CH // 16):
                rid = lax.iota(jnp.int32, 16) + (16 * j)
                bufT[d, pl.ds(16 * j, 16)] = plsc.load_gather(
                    bufA, [rid, cid])
            return carry

        lax.fori_loop(0, D, drow, 0)
        pltpu.sync_copy(bufT, out_hbm.at[l, :, pl.ds(b0 + c * CH, CH)])

        for k in range(CH // 16):
            v = idx_v[l, pl.ds(c * CH + 16 * k, 16)]
            mask_v[l, pl.ds(c * CH + 16 * k, 16)] = (
                1 - jnp.minimum(v, 1)).astype(jnp.float32)

    start(0, bufA0, sem0)

    def pair(t2, carry):
        t0 = 2 * t2
        start(t0 + 1, bufA1, sem1)
        finish(t0, bufA0, bufT0, sem0)

        @pl.when(t2 + 1 < NCHUNK // 2)
        def _():
            start(t0 + 2, bufA0, sem0)

        finish(t0 + 1, bufA1, bufT1, sem1)
        return carry

    lax.fori_loop(0, NCHUNK // 2, pair, 0)
    pltpu.sync_copy(mask_v, mask_hbm.at[:, pl.ds(b0, BW)])


@functools.lru_cache(maxsize=1)
def _gather_kernel():
    return pl.kernel(
        _gather_body,
        out_type=(
            jax.ShapeDtypeStruct((L, D, B), jnp.float32),
            jax.ShapeDtypeStruct((LP, B), jnp.float32),
        ),
        mesh=plsc.VectorSubcoreMesh(
            core_axis_name="c", subcore_axis_name="s",
            num_cores=NC, num_subcores=NS,
        ),
        scratch_types=[
            pltpu.VMEM((LP, BW), jnp.int32),
            pltpu.VMEM((CH, 2 * D), jnp.float32),
            pltpu.VMEM((CH, 2 * D), jnp.float32),
            pltpu.VMEM((D, CH), jnp.float32),
            pltpu.VMEM((D, CH), jnp.float32),
            pltpu.VMEM((LP, BW), jnp.float32),
            pltpu.SemaphoreType.DMA,
            pltpu.SemaphoreType.DMA,
        ],
        compiler_params=pltpu.CompilerParams(
            use_tc_tiling_on_sc=True, needs_layout_passes=False),
    )


def kernel(x, table, W, b):
    tw = _table_times_w(table.T, W, b)
    xp = jnp.pad(x.astype(jnp.int32), ((0, 0), (0, LP - L))).T
    out3, mask_t = _gather_kernel()(xp, tw)
    out = out3.transpose(2, 0, 1)
    mask = mask_t[:L].transpose(1, 0)
    return out, mask


# speedup vs baseline: 1.1144x; 1.1144x over previous
import functools

import jax
import jax.numpy as jnp
import numpy as np
from jax import lax
from jax.experimental import pallas as pl
from jax.experimental.pallas import tpu as pltpu
from jax.experimental.pallas import tpu_sc as plsc

B, L = 16384, 50
VOCAB, D = 1000000, 64
LP = 64
NC, NS = 2, 16
NW = NC * NS
BW = B // NW
CH = 128
NCHUNK = L * (BW // CH)
MM_BLK = 8192
_IOTA16 = np.arange(16, dtype=np.int32)


def _mm_body(tt_ref, w_ref, b_ref, o_ref):
    tw = lax.dot_general(
        tt_ref[...], w_ref[...],
        dimension_numbers=(((0,), (0,)), ((), ())),
        preferred_element_type=jnp.float32,
    ) + b_ref[...]
    o_ref[:, 0:D] = tw
    o_ref[:, D:2 * D] = tw


def _table_times_w(table_t, W, b):
    return pl.pallas_call(
        _mm_body,
        grid=(pl.cdiv(VOCAB, MM_BLK),),
        in_specs=[
            pl.BlockSpec((D, MM_BLK), lambda i: (0, i)),
            pl.BlockSpec((D, D), lambda i: (0, 0)),
            pl.BlockSpec((1, D), lambda i: (0, 0)),
        ],
        out_specs=pl.BlockSpec((MM_BLK, 2 * D), lambda i: (i, 0)),
        out_shape=jax.ShapeDtypeStruct((VOCAB, 2 * D), jnp.float32),
    )(table_t, W, b.reshape(1, D))


def _gather_body(xp_hbm, tw_hbm, out_hbm, mask_hbm,
                 idx_v, bufA0, bufA1, bufT0, bufT1, mask_v, diag_v,
                 sem0, sem1):
    wid = lax.axis_index("s") * NC + lax.axis_index("c")
    b0 = wid * BW

    for g in range(D // 16):
        for s in range(16):
            diag_v[g * 16 + s, :] = (
                (lax.iota(jnp.int32, 16) + s) & 15) + 16 * g

    pltpu.sync_copy(xp_hbm.at[:, pl.ds(b0, BW)], idx_v)

    def start(t, buf, sem):
        l, c = t // 4, t % 4
        pltpu.make_async_copy(
            tw_hbm.at[idx_v.at[l, pl.ds(c * CH, CH)]], buf, sem).start()

    def finish(t, bufA, bufT, sem):
        l, c = t // 4, t % 4
        pltpu.make_async_copy(
            tw_hbm.at[idx_v.at[l, pl.ds(c * CH, CH)]], bufA, sem).wait()

        def kblk(k, carry):
            rowv = lax.iota(jnp.int32, 16) + 16 * k
            for g in range(D // 16):
                for s in range(16):
                    diag = diag_v[g * 16 + s, :]
                    vals = plsc.load_gather(bufA, [rowv, diag])
                    plsc.store_scatter(bufT, [diag, rowv], vals)
            return carry

        lax.fori_loop(0, CH // 16, kblk, 0)
        pltpu.sync_copy(bufT, out_hbm.at[l, :, pl.ds(b0 + c * CH, CH)])

        for k in range(CH // 16):
            v = idx_v[l, pl.ds(c * CH + 16 * k, 16)]
            mask_v[l, pl.ds(c * CH + 16 * k, 16)] = (
                1 - jnp.minimum(v, 1)).astype(jnp.float32)

    start(0, bufA0, sem0)

    def pair(t2, carry):
        t0 = 2 * t2
        start(t0 + 1, bufA1, sem1)
        finish(t0, bufA0, bufT0, sem0)

        @pl.when(t2 + 1 < NCHUNK // 2)
        def _():
            start(t0 + 2, bufA0, sem0)

        finish(t0 + 1, bufA1, bufT1, sem1)
        return carry

    lax.fori_loop(0, NCHUNK // 2, pair, 0)
    pltpu.sync_copy(mask_v, mask_hbm.at[:, pl.ds(b0, BW)])


@functools.lru_cache(maxsize=1)
def _gather_kernel():
    return pl.kernel(
        _gather_body,
        out_type=(
            jax.ShapeDtypeStruct((L, D, B), jnp.float32),
            jax.ShapeDtypeStruct((LP, B), jnp.float32),
        ),
        mesh=plsc.VectorSubcoreMesh(
            core_axis_name="c", subcore_axis_name="s",
            num_cores=NC, num_subcores=NS,
        ),
        scratch_types=[
            pltpu.VMEM((LP, BW), jnp.int32),
            pltpu.VMEM((CH, 2 * D), jnp.float32),
            pltpu.VMEM((CH, 2 * D), jnp.float32),
            pltpu.VMEM((D, CH), jnp.float32),
            pltpu.VMEM((D, CH), jnp.float32),
            pltpu.VMEM((LP, BW), jnp.float32),
            pltpu.VMEM((D, 16), jnp.int32),
            pltpu.SemaphoreType.DMA,
            pltpu.SemaphoreType.DMA,
        ],
        compiler_params=pltpu.CompilerParams(
            use_tc_tiling_on_sc=True, needs_layout_passes=False),
    )


def kernel(x, table, W, b):
    tw = _table_times_w(table.T, W, b)
    xp = jnp.pad(x.astype(jnp.int32), ((0, 0), (0, LP - L))).T
    out3, mask_t = _gather_kernel()(xp, tw)
    out = out3.transpose(2, 0, 1)
    mask = mask_t[:L].transpose(1, 0)
    return out, mask


# speedup vs baseline: 2.5671x; 2.3035x over previous
import functools

import jax
import jax.numpy as jnp
import numpy as np
from jax import lax
from jax.experimental import pallas as pl
from jax.experimental.pallas import tpu as pltpu
from jax.experimental.pallas import tpu_sc as plsc

B, L = 16384, 50
VOCAB, D = 1000000, 64
LP = 64
NC, NS = 2, 16
NW = NC * NS
BW = B // NW
CH = 128
NCHUNK = L * (BW // CH)
MM_BLK = 8192
_IOTA16 = np.arange(16, dtype=np.int32)


def _mm_body(tt_ref, w_ref, b_ref, o_ref):
    tw = lax.dot_general(
        tt_ref[...], w_ref[...],
        dimension_numbers=(((0,), (0,)), ((), ())),
        preferred_element_type=jnp.float32,
    ) + b_ref[...]
    o_ref[:, 0:D] = tw
    o_ref[:, D:2 * D] = tw


def _table_times_w(table_t, W, b):
    return pl.pallas_call(
        _mm_body,
        grid=(pl.cdiv(VOCAB, MM_BLK),),
        in_specs=[
            pl.BlockSpec((D, MM_BLK), lambda i: (0, i)),
            pl.BlockSpec((D, D), lambda i: (0, 0)),
            pl.BlockSpec((1, D), lambda i: (0, 0)),
        ],
        out_specs=pl.BlockSpec((MM_BLK, 2 * D), lambda i: (i, 0)),
        out_shape=jax.ShapeDtypeStruct((VOCAB, 2 * D), jnp.float32),
    )(table_t, W, b.reshape(1, D))


def _gather_body(xp_hbm, tw_hbm, out_hbm, mask_hbm,
                 idx_v, bufA0, bufA1, bufT0, bufT1, mask_v, diag_v,
                 sem0, sem1):
    wid = lax.axis_index("s") * NC + lax.axis_index("c")
    b0 = wid * BW

    for g in range(D // 16):
        for s in range(16):
            diag_v[g * 16 + s, :] = (
                (lax.iota(jnp.int32, 16) + s) & 15) + 16 * g

    pltpu.sync_copy(xp_hbm.at[:, pl.ds(b0, BW)], idx_v)

    def start(t, buf, sem):
        l, c = t // 4, t % 4
        pltpu.make_async_copy(
            tw_hbm.at[idx_v.at[l, pl.ds(c * CH, CH)]], buf, sem).start()

    def finish(t, bufA, bufT, sem):
        l, c = t // 4, t % 4
        pltpu.make_async_copy(
            tw_hbm.at[idx_v.at[l, pl.ds(c * CH, CH)]], bufA, sem).wait()

        def kblk(k, carry):
            rowv = lax.iota(jnp.int32, 16) + 16 * k
            for g in range(D // 16):
                diags = [diag_v[g * 16 + s, :] for s in range(16)]
                vals = [plsc.load_gather(bufA, [rowv, diags[s]])
                        for s in range(16)]
                for s in range(16):
                    plsc.store_scatter(bufT, [diags[s], rowv], vals[s])
            return carry

        lax.fori_loop(0, CH // 16, kblk, 0)
        pltpu.sync_copy(bufT, out_hbm.at[l, :, pl.ds(b0 + c * CH, CH)])

        for k in range(CH // 16):
            v = idx_v[l, pl.ds(c * CH + 16 * k, 16)]
            mask_v[l, pl.ds(c * CH + 16 * k, 16)] = (
                1 - jnp.minimum(v, 1)).astype(jnp.float32)

    start(0, bufA0, sem0)

    def pair(t2, carry):
        t0 = 2 * t2
        start(t0 + 1, bufA1, sem1)
        finish(t0, bufA0, bufT0, sem0)

        @pl.when(t2 + 1 < NCHUNK // 2)
        def _():
            start(t0 + 2, bufA0, sem0)

        finish(t0 + 1, bufA1, bufT1, sem1)
        return carry

    lax.fori_loop(0, NCHUNK // 2, pair, 0)
    pltpu.sync_copy(mask_v, mask_hbm.at[:, pl.ds(b0, BW)])


@functools.lru_cache(maxsize=1)
def _gather_kernel():
    return pl.kernel(
        _gather_body,
        out_type=(
            jax.ShapeDtypeStruct((L, D, B), jnp.float32),
            jax.ShapeDtypeStruct((LP, B), jnp.float32),
        ),
        mesh=plsc.VectorSubcoreMesh(
            core_axis_name="c", subcore_axis_name="s",
            num_cores=NC, num_subcores=NS,
        ),
        scratch_types=[
            pltpu.VMEM((LP, BW), jnp.int32),
            pltpu.VMEM((CH, 2 * D), jnp.float32),
            pltpu.VMEM((CH, 2 * D), jnp.float32),
            pltpu.VMEM((D, CH), jnp.float32),
            pltpu.VMEM((D, CH), jnp.float32),
            pltpu.VMEM((LP, BW), jnp.float32),
            pltpu.VMEM((D, 16), jnp.int32),
            pltpu.SemaphoreType.DMA,
            pltpu.SemaphoreType.DMA,
        ],
        compiler_params=pltpu.CompilerParams(
            use_tc_tiling_on_sc=True, needs_layout_passes=False,
            disable_bounds_checks=True),
    )


def kernel(x, table, W, b):
    tw = _table_times_w(table.T, W, b)
    xp = jnp.pad(x.astype(jnp.int32), ((0, 0), (0, LP - L))).T
    out3, mask_t = _gather_kernel()(xp, tw)
    out = out3.transpose(2, 0, 1)
    mask = mask_t[:L].transpose(1, 0)
    return out, mask


# speedup vs baseline: 2.7117x; 1.0563x over previous
import functools

import jax
import jax.numpy as jnp
import numpy as np
from jax import lax
from jax.experimental import pallas as pl
from jax.experimental.pallas import tpu as pltpu
from jax.experimental.pallas import tpu_sc as plsc

B, L = 16384, 50
VOCAB, D = 1000000, 64
LP = 64
NC, NS = 2, 16
NW = NC * NS
BW = B // NW
CH = 128
NCHUNK = L * (BW // CH)
MM_BLK = 16384
_IOTA16 = np.arange(16, dtype=np.int32)


def _mm_body(tt_ref, w_ref, b_ref, o_ref):
    tw = lax.dot_general(
        tt_ref[...], w_ref[...],
        dimension_numbers=(((0,), (0,)), ((), ())),
        preferred_element_type=jnp.float32,
    ) + b_ref[...]
    o_ref[:, 0:D] = tw
    o_ref[:, D:2 * D] = tw


def _table_times_w(table_t, W, b):
    return pl.pallas_call(
        _mm_body,
        grid=(pl.cdiv(VOCAB, MM_BLK),),
        in_specs=[
            pl.BlockSpec((D, MM_BLK), lambda i: (0, i)),
            pl.BlockSpec((D, D), lambda i: (0, 0)),
            pl.BlockSpec((1, D), lambda i: (0, 0)),
        ],
        out_specs=pl.BlockSpec((MM_BLK, 2 * D), lambda i: (i, 0)),
        out_shape=jax.ShapeDtypeStruct((VOCAB, 2 * D), jnp.float32),
    )(table_t, W, b.reshape(1, D))


def _gather_body(xp_hbm, tw_hbm, out_hbm, mask_hbm,
                 idx_v, bufA0, bufA1, bufT0, bufT1, mask_v, diag_v,
                 sem0, sem1):
    wid = lax.axis_index("s") * NC + lax.axis_index("c")
    b0 = wid * BW

    for g in range(D // 16):
        for s in range(16):
            diag_v[g * 16 + s, :] = (
                (lax.iota(jnp.int32, 16) + s) & 15) + 16 * g

    pltpu.sync_copy(xp_hbm.at[:, pl.ds(b0, BW)], idx_v)

    def start(t, buf, sem):
        l, c = t // 4, t % 4
        pltpu.make_async_copy(
            tw_hbm.at[idx_v.at[l, pl.ds(c * CH, CH)]], buf, sem).start()

    def finish(t, bufA, bufT, sem):
        l, c = t // 4, t % 4
        pltpu.make_async_copy(
            tw_hbm.at[idx_v.at[l, pl.ds(c * CH, CH)]], bufA, sem).wait()

        def kblk(k, carry):
            rowv = lax.iota(jnp.int32, 16) + 16 * k
            for g in range(D // 16):
                diags = [diag_v[g * 16 + s, :] for s in range(16)]
                vals = [plsc.load_gather(bufA, [rowv, diags[s]])
                        for s in range(16)]
                for s in range(16):
                    plsc.store_scatter(bufT, [diags[s], rowv], vals[s])
            return carry

        lax.fori_loop(0, CH // 16, kblk, 0)
        pltpu.sync_copy(bufT, out_hbm.at[l, :, pl.ds(b0 + c * CH, CH)])

        for k in range(CH // 16):
            v = idx_v[l, pl.ds(c * CH + 16 * k, 16)]
            mask_v[l, pl.ds(c * CH + 16 * k, 16)] = (
                1 - jnp.minimum(v, 1)).astype(jnp.float32)

    start(0, bufA0, sem0)

    def pair(t2, carry):
        t0 = 2 * t2
        start(t0 + 1, bufA1, sem1)
        finish(t0, bufA0, bufT0, sem0)

        @pl.when(t2 + 1 < NCHUNK // 2)
        def _():
            start(t0 + 2, bufA0, sem0)

        finish(t0 + 1, bufA1, bufT1, sem1)
        return carry

    lax.fori_loop(0, NCHUNK // 2, pair, 0)
    pltpu.sync_copy(mask_v, mask_hbm.at[:, pl.ds(b0, BW)])


@functools.lru_cache(maxsize=1)
def _gather_kernel():
    return pl.kernel(
        _gather_body,
        out_type=(
            jax.ShapeDtypeStruct((L, D, B), jnp.float32),
            jax.ShapeDtypeStruct((LP, B), jnp.float32),
        ),
        mesh=plsc.VectorSubcoreMesh(
            core_axis_name="c", subcore_axis_name="s",
            num_cores=NC, num_subcores=NS,
        ),
        scratch_types=[
            pltpu.VMEM((LP, BW), jnp.int32),
            pltpu.VMEM((CH, 2 * D), jnp.float32),
            pltpu.VMEM((CH, 2 * D), jnp.float32),
            pltpu.VMEM((D, CH), jnp.float32),
            pltpu.VMEM((D, CH), jnp.float32),
            pltpu.VMEM((LP, BW), jnp.float32),
            pltpu.VMEM((D, 16), jnp.int32),
            pltpu.SemaphoreType.DMA,
            pltpu.SemaphoreType.DMA,
        ],
        compiler_params=pltpu.CompilerParams(
            use_tc_tiling_on_sc=True, needs_layout_passes=False,
            disable_bounds_checks=True),
    )


def kernel(x, table, W, b):
    tw = _table_times_w(table.T, W, b)
    xp = jnp.pad(x.astype(jnp.int32), ((0, 0), (0, LP - L))).T
    out3, mask_t = _gather_kernel()(xp, tw)
    out = out3.transpose(2, 0, 1)
    mask = mask_t[:L].transpose(1, 0)
    return out, mask


# speedup vs baseline: 2.8987x; 1.0690x over previous
import functools

import jax
import jax.numpy as jnp
import numpy as np
from jax import lax
from jax.experimental import pallas as pl
from jax.experimental.pallas import tpu as pltpu
from jax.experimental.pallas import tpu_sc as plsc

B, L = 16384, 50
VOCAB, D = 1000000, 64
LP = 56
NC, NS = 2, 16
NW = NC * NS
BW = B // NW
CH = 128
NCHUNK = L * (BW // CH)
MM_BLK = 8192
NB2 = 62
HALF = NB2 * MM_BLK
SPLIT = (NB2 - 1) * MM_BLK


def _dot_t(tt, w, bb):
    return lax.dot_general(
        tt, w,
        dimension_numbers=(((0,), (0,)), ((), ())),
        preferred_element_type=jnp.float32,
    ) + bb


def _mm_body(t1_ref, t2_ref, w_ref, b_ref, o_ref):
    o_ref[:, 0:D] = _dot_t(t1_ref[...], w_ref[...], b_ref[...])
    o_ref[:, D:2 * D] = _dot_t(t2_ref[...], w_ref[...], b_ref[...])


def _table_times_w(table_t, W, b):
    return pl.pallas_call(
        _mm_body,
        grid=(NB2,),
        in_specs=[
            pl.BlockSpec((D, MM_BLK), lambda i: (0, i)),
            pl.BlockSpec((D, MM_BLK), lambda i: (0, i + NB2 - 1)),
            pl.BlockSpec((D, D), lambda i: (0, 0)),
            pl.BlockSpec((1, D), lambda i: (0, 0)),
        ],
        out_specs=pl.BlockSpec((MM_BLK, 2 * D), lambda i: (i, 0)),
        out_shape=jax.ShapeDtypeStruct((HALF, 2 * D), jnp.float32),
    )(table_t, table_t, W, b.reshape(1, D))


def _gather_body(xp_hbm, tw_hbm, out_hbm, mask_hbm,
                 idx_v, idxh_v, bufA0, bufA1, bufT, mask_v, diag_v,
                 sem0, sem1):
    wid = lax.axis_index("s") * NC + lax.axis_index("c")
    b0 = wid * BW

    for g in range(D // 16):
        for s in range(16):
            diag_v[g, pl.ds(16 * s, 16)] = (
                (lax.iota(jnp.int32, 16) + s) & 15) + 16 * g

    pltpu.sync_copy(xp_hbm.at[:, pl.ds(b0, BW)], idx_v)

    def halve(r, carry):
        for k in range(BW // 16):
            v = idx_v[r, pl.ds(16 * k, 16)]
            ge = 1 + ((v - SPLIT) >> 31)
            idxh_v[r, pl.ds(16 * k, 16)] = v - ge * SPLIT
        return carry

    lax.fori_loop(0, LP, halve, 0)

    def start(t, buf, sem):
        l, c = t // 4, t % 4
        pltpu.make_async_copy(
            tw_hbm.at[idxh_v.at[l, pl.ds(c * CH, CH)]], buf, sem).start()

    def finish(t, bufA, sem):
        l, c = t // 4, t % 4
        pltpu.make_async_copy(
            tw_hbm.at[idxh_v.at[l, pl.ds(c * CH, CH)]], bufA, sem).wait()

        def kblk(k, carry):
            rowv = lax.iota(jnp.int32, 16) + 16 * k
            v = idx_v[l, pl.ds(c * CH + 16 * k, 16)]
            par = (1 + ((v - SPLIT) >> 31)) << 6
            for g in range(D // 16):
                diags = [diag_v[g, pl.ds(16 * s, 16)] for s in range(16)]
                vals = [plsc.load_gather(bufA, [rowv, diags[s] + par])
                        for s in range(16)]
                for s in range(16):
                    plsc.store_scatter(bufT, [diags[s], rowv], vals[s])
            return carry

        lax.fori_loop(0, CH // 16, kblk, 0)
        pltpu.sync_copy(bufT, out_hbm.at[l, :, pl.ds(b0 + c * CH, CH)])

        for k in range(CH // 16):
            v = idx_v[l, pl.ds(c * CH + 16 * k, 16)]
            mask_v[l, pl.ds(c * CH + 16 * k, 16)] = (
                1 - jnp.minimum(v, 1)).astype(jnp.float32)

    start(0, bufA0, sem0)

    def pair(t2, carry):
        t0 = 2 * t2
        start(t0 + 1, bufA1, sem1)
        finish(t0, bufA0, sem0)

        @pl.when(t2 + 1 < NCHUNK // 2)
        def _():
            start(t0 + 2, bufA0, sem0)

        finish(t0 + 1, bufA1, sem1)
        return carry

    lax.fori_loop(0, NCHUNK // 2, pair, 0)
    pltpu.sync_copy(mask_v, mask_hbm.at[:, pl.ds(b0, BW)])


@functools.lru_cache(maxsize=1)
def _gather_kernel():
    return pl.kernel(
        _gather_body,
        out_type=(
            jax.ShapeDtypeStruct((L, D, B), jnp.float32),
            jax.ShapeDtypeStruct((LP, B), jnp.float32),
        ),
        mesh=plsc.VectorSubcoreMesh(
            core_axis_name="c", subcore_axis_name="s",
            num_cores=NC, num_subcores=NS,
        ),
        scratch_types=[
            pltpu.VMEM((LP, BW), jnp.int32),
            pltpu.VMEM((LP, BW), jnp.int32),
            pltpu.VMEM((CH, 2 * D), jnp.float32),
            pltpu.VMEM((CH, 2 * D), jnp.float32),
            pltpu.VMEM((D, CH), jnp.float32),
            pltpu.VMEM((LP, BW), jnp.float32),
            pltpu.VMEM((D // 16, 16 * 16), jnp.int32),
            pltpu.SemaphoreType.DMA,
            pltpu.SemaphoreType.DMA,
        ],
        compiler_params=pltpu.CompilerParams(
            use_tc_tiling_on_sc=True, needs_layout_passes=False,
            disable_bounds_checks=True),
    )


def kernel(x, table, W, b):
    tw = _table_times_w(table.T, W, b)
    xp = jnp.pad(x.astype(jnp.int32), ((0, 0), (0, LP - L))).T
    out3, mask_t = _gather_kernel()(xp, tw)
    out = out3.transpose(2, 0, 1)
    mask = mask_t[:L].transpose(1, 0)
    return out, mask


# speedup vs baseline: 2.9316x; 1.0113x over previous
import functools

import jax
import jax.numpy as jnp
import numpy as np
from jax import lax
from jax.experimental import pallas as pl
from jax.experimental.pallas import tpu as pltpu
from jax.experimental.pallas import tpu_sc as plsc

B, L = 16384, 50
VOCAB, D = 1000000, 64
LP = 56
NC, NS = 2, 16
NW = NC * NS
BW = B // NW
CH = 128
NCHUNK = L * (BW // CH)
MM_BLK = 16384
NB2 = 32
HALF = NB2 * MM_BLK
SPLIT = (NB2 - 2) * MM_BLK


def _dot_t(tt, w, bb):
    return lax.dot_general(
        tt, w,
        dimension_numbers=(((0,), (0,)), ((), ())),
        preferred_element_type=jnp.float32,
    ) + bb


def _mm_body(t1_ref, t2_ref, w_ref, b_ref, o_ref):
    o_ref[:, 0:D] = _dot_t(t1_ref[...], w_ref[...], b_ref[...])
    o_ref[:, D:2 * D] = _dot_t(t2_ref[...], w_ref[...], b_ref[...])


def _table_times_w(table_t, W, b):
    return pl.pallas_call(
        _mm_body,
        grid=(NB2,),
        in_specs=[
            pl.BlockSpec((D, MM_BLK), lambda i: (0, i)),
            pl.BlockSpec((D, MM_BLK), lambda i: (0, i + NB2 - 2)),
            pl.BlockSpec((D, D), lambda i: (0, 0)),
            pl.BlockSpec((1, D), lambda i: (0, 0)),
        ],
        out_specs=pl.BlockSpec((MM_BLK, 2 * D), lambda i: (i, 0)),
        out_shape=jax.ShapeDtypeStruct((HALF, 2 * D), jnp.float32),
    )(table_t, table_t, W, b.reshape(1, D))


def _gather_body(xp_hbm, tw_hbm, out_hbm, mask_hbm,
                 idx_v, idxh_v, bufA0, bufA1, bufT, mask_v, diag_v,
                 sem0, sem1):
    wid = lax.axis_index("s") * NC + lax.axis_index("c")
    b0 = wid * BW

    for g in range(D // 16):
        for s in range(16):
            diag_v[g, pl.ds(16 * s, 16)] = (
                (lax.iota(jnp.int32, 16) + s) & 15) + 16 * g

    pltpu.sync_copy(xp_hbm.at[:, pl.ds(b0, BW)], idx_v)

    def halve(r, carry):
        for k in range(BW // 16):
            v = idx_v[r, pl.ds(16 * k, 16)]
            ge = 1 + ((v - SPLIT) >> 31)
            idxh_v[r, pl.ds(16 * k, 16)] = v - ge * SPLIT
        return carry

    lax.fori_loop(0, LP, halve, 0)

    def start(t, buf, sem):
        l, c = t // 4, t % 4
        pltpu.make_async_copy(
            tw_hbm.at[idxh_v.at[l, pl.ds(c * CH, CH)]], buf, sem).start()

    def finish(t, bufA, sem):
        l, c = t // 4, t % 4
        pltpu.make_async_copy(
            tw_hbm.at[idxh_v.at[l, pl.ds(c * CH, CH)]], bufA, sem).wait()

        def kblk(k, carry):
            rowv = lax.iota(jnp.int32, 16) + 16 * k
            v = idx_v[l, pl.ds(c * CH + 16 * k, 16)]
            par = (1 + ((v - SPLIT) >> 31)) << 6
            for g in range(D // 16):
                diags = [diag_v[g, pl.ds(16 * s, 16)] for s in range(16)]
                vals = [plsc.load_gather(bufA, [rowv, diags[s] + par])
                        for s in range(16)]
                for s in range(16):
                    plsc.store_scatter(bufT, [diags[s], rowv], vals[s])
            return carry

        lax.fori_loop(0, CH // 16, kblk, 0)
        pltpu.sync_copy(bufT, out_hbm.at[l, :, pl.ds(b0 + c * CH, CH)])

        for k in range(CH // 16):
            v = idx_v[l, pl.ds(c * CH + 16 * k, 16)]
            mask_v[l, pl.ds(c * CH + 16 * k, 16)] = (
                1 - jnp.minimum(v, 1)).astype(jnp.float32)

    start(0, bufA0, sem0)

    def pair(t2, carry):
        t0 = 2 * t2
        start(t0 + 1, bufA1, sem1)
        finish(t0, bufA0, sem0)

        @pl.when(t2 + 1 < NCHUNK // 2)
        def _():
            start(t0 + 2, bufA0, sem0)

        finish(t0 + 1, bufA1, sem1)
        return carry

    lax.fori_loop(0, NCHUNK // 2, pair, 0)
    pltpu.sync_copy(mask_v, mask_hbm.at[:, pl.ds(b0, BW)])


@functools.lru_cache(maxsize=1)
def _gather_kernel():
    return pl.kernel(
        _gather_body,
        out_type=(
            jax.ShapeDtypeStruct((L, D, B), jnp.float32),
            jax.ShapeDtypeStruct((LP, B), jnp.float32),
        ),
        mesh=plsc.VectorSubcoreMesh(
            core_axis_name="c", subcore_axis_name="s",
            num_cores=NC, num_subcores=NS,
        ),
        scratch_types=[
            pltpu.VMEM((LP, BW), jnp.int32),
            pltpu.VMEM((LP, BW), jnp.int32),
            pltpu.VMEM((CH, 2 * D), jnp.float32),
            pltpu.VMEM((CH, 2 * D), jnp.float32),
            pltpu.VMEM((D, CH), jnp.float32),
            pltpu.VMEM((LP, BW), jnp.float32),
            pltpu.VMEM((D // 16, 16 * 16), jnp.int32),
            pltpu.SemaphoreType.DMA,
            pltpu.SemaphoreType.DMA,
        ],
        compiler_params=pltpu.CompilerParams(
            use_tc_tiling_on_sc=True, needs_layout_passes=False,
            disable_bounds_checks=True),
    )


def kernel(x, table, W, b):
    tw = _table_times_w(table.T, W, b)
    xp = jnp.pad(x.astype(jnp.int32), ((0, 0), (0, LP - L))).T
    out3, mask_t = _gather_kernel()(xp, tw)
    out = out3.transpose(2, 0, 1)
    mask = mask_t[:L].transpose(1, 0)
    return out, mask


# speedup vs baseline: 3.0248x; 1.0318x over previous
import functools

import jax
import jax.numpy as jnp
import numpy as np
from jax import lax
from jax.experimental import pallas as pl
from jax.experimental.pallas import tpu as pltpu
from jax.experimental.pallas import tpu_sc as plsc

B, L = 16384, 50
VOCAB, D = 1000000, 64
LP = 56
NC, NS = 2, 16
NW = NC * NS
BW = B // NW
CH = 128
NCHUNK = L * (BW // CH)
MM_BLK = 16384
NB2 = 32
HALF = NB2 * MM_BLK
SPLIT = (NB2 - 2) * MM_BLK


def _dot_t(tt, w, bb):
    return lax.dot_general(
        tt, w,
        dimension_numbers=(((0,), (0,)), ((), ())),
        preferred_element_type=jnp.float32,
    ) + bb


def _mm_body(t1_ref, t2_ref, w_ref, b_ref, o_ref):
    o_ref[:, 0:D] = _dot_t(t1_ref[...], w_ref[...], b_ref[...])
    o_ref[:, D:2 * D] = _dot_t(t2_ref[...], w_ref[...], b_ref[...])


def _table_times_w(table_t, W, b):
    return pl.pallas_call(
        _mm_body,
        grid=(NB2,),
        in_specs=[
            pl.BlockSpec((D, MM_BLK), lambda i: (0, i)),
            pl.BlockSpec((D, MM_BLK), lambda i: (0, i + NB2 - 2)),
            pl.BlockSpec((D, D), lambda i: (0, 0)),
            pl.BlockSpec((1, D), lambda i: (0, 0)),
        ],
        out_specs=pl.BlockSpec((MM_BLK, 2 * D), lambda i: (i, 0)),
        out_shape=jax.ShapeDtypeStruct((HALF, 2 * D), jnp.float32),
    )(table_t, table_t, W, b.reshape(1, D))


def _gather_body(xp_hbm, tw_hbm, out_hbm, mask_hbm,
                 idx_v, idxh_v, bufA0, bufA1, bufT, mask_v, diag_v,
                 sem0, sem1, sem_out):
    wid = lax.axis_index("s") * NC + lax.axis_index("c")
    b0 = wid * BW

    for g in range(D // 16):
        for s in range(16):
            diag_v[g, pl.ds(16 * s, 16)] = (
                (lax.iota(jnp.int32, 16) + s) & 15) + 16 * g

    pltpu.sync_copy(xp_hbm.at[:, pl.ds(b0, BW)], idx_v)

    def halve(r, carry):
        for k in range(BW // 16):
            v = idx_v[r, pl.ds(16 * k, 16)]
            ge = 1 + ((v - SPLIT) >> 31)
            idxh_v[r, pl.ds(16 * k, 16)] = v - ge * SPLIT
        return carry

    lax.fori_loop(0, LP, halve, 0)

    def start(t, buf, sem):
        l, c = t // 4, t % 4
        pltpu.make_async_copy(
            tw_hbm.at[idxh_v.at[l, pl.ds(c * CH, CH)]], buf, sem).start()

    def finish(t, bufA, sem):
        l, c = t // 4, t % 4
        pltpu.make_async_copy(
            tw_hbm.at[idxh_v.at[l, pl.ds(c * CH, CH)]], bufA, sem).wait()

        @pl.when(t > 0)
        def _():
            pltpu.make_async_copy(
                bufT, out_hbm.at[l, :, pl.ds(b0 + c * CH, CH)],
                sem_out).wait()

        def kblk(k, carry):
            rowv = lax.iota(jnp.int32, 16) + 16 * k
            v = idx_v[l, pl.ds(c * CH + 16 * k, 16)]
            par = (1 + ((v - SPLIT) >> 31)) << 6
            for g in range(D // 16):
                diags = [diag_v[g, pl.ds(16 * s, 16)] for s in range(16)]
                vals = [plsc.load_gather(bufA, [rowv, diags[s] + par])
                        for s in range(16)]
                for s in range(16):
                    plsc.store_scatter(bufT, [diags[s], rowv], vals[s])
            return carry

        lax.fori_loop(0, CH // 16, kblk, 0)
        pltpu.make_async_copy(
            bufT, out_hbm.at[l, :, pl.ds(b0 + c * CH, CH)], sem_out).start()

        for k in range(CH // 16):
            v = idx_v[l, pl.ds(c * CH + 16 * k, 16)]
            mask_v[l, pl.ds(c * CH + 16 * k, 16)] = (
                1 - jnp.minimum(v, 1)).astype(jnp.float32)

    start(0, bufA0, sem0)

    def pair(t2, carry):
        t0 = 2 * t2
        start(t0 + 1, bufA1, sem1)
        finish(t0, bufA0, sem0)

        @pl.when(t2 + 1 < NCHUNK // 2)
        def _():
            start(t0 + 2, bufA0, sem0)

        finish(t0 + 1, bufA1, sem1)
        return carry

    lax.fori_loop(0, NCHUNK // 2, pair, 0)
    pltpu.make_async_copy(
        bufT, out_hbm.at[L - 1, :, pl.ds(b0 + (BW - CH), CH)],
        sem_out).wait()
    pltpu.sync_copy(mask_v, mask_hbm.at[:, pl.ds(b0, BW)])


@functools.lru_cache(maxsize=1)
def _gather_kernel():
    return pl.kernel(
        _gather_body,
        out_type=(
            jax.ShapeDtypeStruct((L, D, B), jnp.float32),
            jax.ShapeDtypeStruct((LP, B), jnp.float32),
        ),
        mesh=plsc.VectorSubcoreMesh(
            core_axis_name="c", subcore_axis_name="s",
            num_cores=NC, num_subcores=NS,
        ),
        scratch_types=[
            pltpu.VMEM((LP, BW), jnp.int32),
            pltpu.VMEM((LP, BW), jnp.int32),
            pltpu.VMEM((CH, 2 * D), jnp.float32),
            pltpu.VMEM((CH, 2 * D), jnp.float32),
            pltpu.VMEM((D, CH), jnp.float32),
            pltpu.VMEM((LP, BW), jnp.float32),
            pltpu.VMEM((D // 16, 16 * 16), jnp.int32),
            pltpu.SemaphoreType.DMA,
            pltpu.SemaphoreType.DMA,
            pltpu.SemaphoreType.DMA,
        ],
        compiler_params=pltpu.CompilerParams(
            use_tc_tiling_on_sc=True, needs_layout_passes=False,
            disable_bounds_checks=True),
    )


def kernel(x, table, W, b):
    tw = _table_times_w(table.T, W, b)
    xp = jnp.pad(x.astype(jnp.int32), ((0, 0), (0, LP - L))).T
    out3, mask_t = _gather_kernel()(xp, tw)
    out = out3.transpose(2, 0, 1)
    mask = mask_t[:L].transpose(1, 0)
    return out, mask
